# baseline (speedup 1.0000x reference)
import functools

import jax
import jax.numpy as jnp
from jax.experimental import pallas as pl

_D = 128
_K = 16
_SQ = 0.7071067811865476


def _gelu_scaled(y):
    z = _SQ * y
    return z + z * jax.lax.erf(y)


def _body(x0_ref, x1_ref, ws0_ref, ws1_ref, wa1_ref, out_ref):
    ws0 = ws0_ref[...]
    blk = x0_ref.shape[0]
    g = _gelu_scaled(
        jnp.dot(x1_ref[...], ws0, preferred_element_type=jnp.float32)
    )
    s = g.reshape(blk, _K, _D).sum(axis=1)
    a = _gelu_scaled(
        jnp.dot(x0_ref[...], ws0, preferred_element_type=jnp.float32)
    )
    out_ref[:, :_D] = jnp.dot(a, ws1_ref[...], preferred_element_type=jnp.float32)
    out_ref[:, _D:] = jnp.dot(s, wa1_ref[...], preferred_element_type=jnp.float32)


@functools.partial(jax.jit, static_argnames=("blk",))
def _run(x0, x1, w_self0, w_self1, w_agg1, blk=512):
    n0 = x0.shape[0]
    grid = (n0 // blk,)
    ws0p = w_self0 * jnp.float32(_SQ)
    return pl.pallas_call(
        _body,
        grid=grid,
        in_specs=[
            pl.BlockSpec((blk, _D), lambda i: (i, 0)),
            pl.BlockSpec((blk * _K, _D), lambda i: (i, 0)),
            pl.BlockSpec((_D, _D), lambda i: (0, 0)),
            pl.BlockSpec((_D, _D), lambda i: (0, 0)),
            pl.BlockSpec((_D, _D), lambda i: (0, 0)),
        ],
        out_specs=pl.BlockSpec((blk, 2 * _D), lambda i: (i, 0)),
        out_shape=jax.ShapeDtypeStruct((n0, 2 * _D), jnp.float32),
    )(x0, x1, ws0p, w_self1, w_agg1)


def kernel(x0, x1, x2, W_self0, W_agg0, W_self1, W_agg1):
    del x2, W_agg0
    return _run(x0, x1, W_self0, W_self1, W_agg1)


# speedup vs baseline: 1.1633x; 1.1633x over previous
import functools

import jax
import jax.numpy as jnp
from jax.experimental import pallas as pl

_D = 128
_K = 16
_SQ = 0.7071067811865476


def _gelu_scaled(y):
    z = _SQ * y
    return z + z * jax.lax.erf(y)


def _body(x0_ref, x1_ref, ws0_ref, ws1_ref, wa1_ref, out_ref):
    ws0 = ws0_ref[...] * _SQ
    blk = x0_ref.shape[0]
    g = _gelu_scaled(
        jnp.dot(x1_ref[...], ws0, preferred_element_type=jnp.float32)
    )
    s = g.reshape(blk, _K, _D).sum(axis=1)
    a = _gelu_scaled(
        jnp.dot(x0_ref[...], ws0, preferred_element_type=jnp.float32)
    )
    out_ref[:, :_D] = jnp.dot(a, ws1_ref[...], preferred_element_type=jnp.float32)
    out_ref[:, _D:] = jnp.dot(s, wa1_ref[...], preferred_element_type=jnp.float32)


@functools.partial(jax.jit, static_argnames=("blk",))
def _run(x0, x1, w_self0, w_self1, w_agg1, blk=1024):
    n0 = x0.shape[0]
    grid = (n0 // blk,)
    return pl.pallas_call(
        _body,
        grid=grid,
        in_specs=[
            pl.BlockSpec((blk, _D), lambda i: (i, 0)),
            pl.BlockSpec((blk * _K, _D), lambda i: (i, 0)),
            pl.BlockSpec((_D, _D), lambda i: (0, 0)),
            pl.BlockSpec((_D, _D), lambda i: (0, 0)),
            pl.BlockSpec((_D, _D), lambda i: (0, 0)),
        ],
        out_specs=pl.BlockSpec((blk, 2 * _D), lambda i: (i, 0)),
        out_shape=jax.ShapeDtypeStruct((n0, 2 * _D), jnp.float32),
    )(x0, x1, w_self0, w_self1, w_agg1)


def kernel(x0, x1, x2, W_self0, W_agg0, W_self1, W_agg1):
    del x2, W_agg0
    return _run(x0, x1, W_self0, W_self1, W_agg1)
